# baseline (speedup 1.0000x reference)
import jax
import jax.numpy as jnp
from jax.experimental import pallas as pl


def _prep_body(edge_ref, edgeT_ref, A_ref, mask_ref, M1_ref):
    S = A_ref.shape[0]
    dst = edge_ref[1:2, :]
    srcT = edgeT_ref[:, 0:1]
    row_ids = jax.lax.broadcasted_iota(jnp.int32, (S, 1), 0)
    col_ids = jax.lax.broadcasted_iota(jnp.int32, (1, S), 1)
    dst_oh = (row_ids == dst).astype(jnp.float32)
    src_oh = (srcT == col_ids).astype(jnp.float32)
    counts = jnp.dot(dst_oh, src_oh, preferred_element_type=jnp.float32)
    A = (counts > 0.5).astype(jnp.float32)
    eye = (row_ids == col_ids).astype(jnp.float32)
    mask = jnp.minimum(A + eye, 1.0)
    A_ref[...] = A
    mask_ref[...] = mask
    M1_ref[...] = jnp.dot(A, mask, preferred_element_type=jnp.float32)


def _stage1_body(x_ref, mask_ref, m1_ref, w1s_ref, w1n_ref, b1_ref, out_ref):
    Ti = out_ref.shape[0]
    x = x_ref[...]
    w1s = w1s_ref[...]
    w1n = w1n_ref[...]
    b1 = b1_ref[...]
    t1 = (mask_ref[...][:, :, None] * w1s[None, :, :]
          + m1_ref[...][:, :, None] * w1n[None, :, :])
    for i in range(Ti):
        h = jnp.dot(x, t1[i], preferred_element_type=jnp.float32) + b1
        out_ref[i] = jax.nn.relu(h)


def _stage2_body(h1_ref, A_ref, w2s_ref, w2n_ref, b2_ref, wr_ref, bout_ref,
                 out_ref):
    S = A_ref.shape[0]
    H = w2s_ref.shape[0]
    Bt = h1_ref.shape[1] // H
    hblk = h1_ref[...]
    agg = jnp.dot(A_ref[...], hblk, preferred_element_type=jnp.float32)
    w2s = w2s_ref[...]
    w2n = w2n_ref[...]
    b2 = b2_ref[...]
    wr = wr_ref[...]
    bo = bout_ref[0, 0]
    for b in range(Bt):
        hb = hblk[:, b * H:(b + 1) * H]
        ab = agg[:, b * H:(b + 1) * H]
        z = jax.nn.relu(
            jnp.dot(hb, w2s, preferred_element_type=jnp.float32)
            + jnp.dot(ab, w2n, preferred_element_type=jnp.float32)
            + b2)
        yb = jnp.sum(z * wr, keepdims=True)
        out_ref[b:b + 1, :] = yb + bo


def kernel(state, action, edge_index, W1_self, W1_neigh, b1,
           W2_self, W2_neigh, b2, W_out, b_out):
    B = state.shape[0]
    S, H = W1_self.shape
    x = jnp.concatenate([state, action], axis=1)
    edgeT = edge_index.T

    A, mask, M1 = pl.pallas_call(
        _prep_body,
        out_shape=[jax.ShapeDtypeStruct((S, S), jnp.float32)] * 3,
    )(edge_index, edgeT)

    Ti = 8
    full = lambda shape: pl.BlockSpec(shape, lambda i: (0,) * len(shape))
    h1 = pl.pallas_call(
        _stage1_body,
        grid=(S // Ti,),
        in_specs=[
            full((B, S)),
            pl.BlockSpec((Ti, S), lambda i: (i, 0)),
            pl.BlockSpec((Ti, S), lambda i: (i, 0)),
            full((S, H)), full((S, H)), full((1, H)),
        ],
        out_specs=pl.BlockSpec((Ti, B, H), lambda i: (i, 0, 0)),
        out_shape=jax.ShapeDtypeStruct((S, B, H), jnp.float32),
    )(x, mask, M1, W1_self, W1_neigh, b1.reshape(1, H))

    Bt = 16
    h1_2d = h1.reshape(S, B * H)
    y = pl.pallas_call(
        _stage2_body,
        grid=(B // Bt,),
        in_specs=[
            pl.BlockSpec((S, Bt * H), lambda j: (0, j)),
            full((S, S)),
            full((H, H)), full((H, H)), full((1, H)),
            full((S, H)), full((1, 1)),
        ],
        out_specs=pl.BlockSpec((Bt, 1), lambda j: (j, 0)),
        out_shape=jax.ShapeDtypeStruct((B, 1), jnp.float32),
    )(h1_2d, A, W2_self, W2_neigh, b2.reshape(1, H), W_out.reshape(S, H),
      b_out.reshape(1, 1))
    return y


# speedup vs baseline: 1.4155x; 1.4155x over previous
import jax
import jax.numpy as jnp
from jax.experimental import pallas as pl


def _prep_body(edge_ref, edgeT_ref, A_ref, mask_ref, M1_ref):
    S = A_ref.shape[0]
    dst = edge_ref[1:2, :]
    srcT = edgeT_ref[:, 0:1]
    row_ids = jax.lax.broadcasted_iota(jnp.int32, (S, 1), 0)
    col_ids = jax.lax.broadcasted_iota(jnp.int32, (1, S), 1)
    dst_oh = (row_ids == dst).astype(jnp.float32)
    src_oh = (srcT == col_ids).astype(jnp.float32)
    counts = jnp.dot(dst_oh, src_oh, preferred_element_type=jnp.float32)
    A = (counts > 0.5).astype(jnp.float32)
    eye = (row_ids == col_ids).astype(jnp.float32)
    mask = jnp.minimum(A + eye, 1.0)
    A_ref[...] = A
    mask_ref[...] = mask
    M1_ref[...] = jnp.dot(A, mask, preferred_element_type=jnp.float32)


def _stage1_body(x_ref, mask_ref, m1_ref, w1s_ref, w1n_ref, b1_ref, out_ref):
    Ti = out_ref.shape[0]
    x = x_ref[...]
    w1s = w1s_ref[...]
    w1n = w1n_ref[...]
    b1 = b1_ref[...]
    t1 = (mask_ref[...][:, :, None] * w1s[None, :, :]
          + m1_ref[...][:, :, None] * w1n[None, :, :])
    for i in range(Ti):
        h = jnp.dot(x, t1[i], preferred_element_type=jnp.float32) + b1
        out_ref[i] = jax.nn.relu(h)


def _stage2_body(h1_ref, A_ref, w2s_ref, w2n_ref, b2_ref, wr_ref, bout_ref,
                 out_ref):
    S, Bt, H = h1_ref.shape
    h3 = h1_ref[...]
    h2 = h3.reshape(S, Bt * H)
    agg = jnp.dot(A_ref[...], h2, preferred_element_type=jnp.float32)
    hr = h3.reshape(S * Bt, H)
    ar = agg.reshape(S * Bt, H)
    z = jax.nn.relu(
        jnp.dot(hr, w2s_ref[...], preferred_element_type=jnp.float32)
        + jnp.dot(ar, w2n_ref[...], preferred_element_type=jnp.float32)
        + b2_ref[...])
    z3 = z.reshape(S, Bt, H)
    y = jnp.sum(z3 * wr_ref[...][:, None, :], axis=(0, 2))
    out_ref[...] = y.reshape(Bt, 1) + bout_ref[0, 0]


def kernel(state, action, edge_index, W1_self, W1_neigh, b1,
           W2_self, W2_neigh, b2, W_out, b_out):
    B = state.shape[0]
    S, H = W1_self.shape
    x = jnp.concatenate([state, action], axis=1)
    edgeT = edge_index.T

    A, mask, M1 = pl.pallas_call(
        _prep_body,
        out_shape=[jax.ShapeDtypeStruct((S, S), jnp.float32)] * 3,
    )(edge_index, edgeT)

    Ti = 8
    full = lambda shape: pl.BlockSpec(shape, lambda i: (0,) * len(shape))
    h1 = pl.pallas_call(
        _stage1_body,
        grid=(S // Ti,),
        in_specs=[
            full((B, S)),
            pl.BlockSpec((Ti, S), lambda i: (i, 0)),
            pl.BlockSpec((Ti, S), lambda i: (i, 0)),
            full((S, H)), full((S, H)), full((1, H)),
        ],
        out_specs=pl.BlockSpec((Ti, B, H), lambda i: (i, 0, 0)),
        out_shape=jax.ShapeDtypeStruct((S, B, H), jnp.float32),
    )(x, mask, M1, W1_self, W1_neigh, b1.reshape(1, H))

    Bt = 32
    y = pl.pallas_call(
        _stage2_body,
        grid=(B // Bt,),
        in_specs=[
            pl.BlockSpec((S, Bt, H), lambda j: (0, j, 0)),
            full((S, S)),
            full((H, H)), full((H, H)), full((1, H)),
            full((S, H)), full((1, 1)),
        ],
        out_specs=pl.BlockSpec((Bt, 1), lambda j: (j, 0)),
        out_shape=jax.ShapeDtypeStruct((B, 1), jnp.float32),
    )(h1, A, W2_self, W2_neigh, b2.reshape(1, H), W_out.reshape(S, H),
      b_out.reshape(1, 1))
    return y


# speedup vs baseline: 1.4608x; 1.0320x over previous
import jax
import jax.numpy as jnp
from jax.experimental import pallas as pl


def _prep_body(edge_ref, edgeT_ref, A_ref, mask_ref, M1_ref):
    S = A_ref.shape[0]
    dst = edge_ref[1:2, :]
    srcT = edgeT_ref[:, 0:1]
    row_ids = jax.lax.broadcasted_iota(jnp.int32, (S, 1), 0)
    col_ids = jax.lax.broadcasted_iota(jnp.int32, (1, S), 1)
    dst_oh = (row_ids == dst).astype(jnp.float32)
    src_oh = (srcT == col_ids).astype(jnp.float32)
    counts = jnp.dot(dst_oh, src_oh, preferred_element_type=jnp.float32)
    A = (counts > 0.5).astype(jnp.float32)
    eye = (row_ids == col_ids).astype(jnp.float32)
    mask = jnp.minimum(A + eye, 1.0)
    A_ref[...] = A
    mask_ref[...] = mask
    M1_ref[...] = jnp.dot(A, mask, preferred_element_type=jnp.float32)


def _stage1_body(x_ref, mask_ref, m1_ref, w1s_ref, w1n_ref, b1_ref, out_ref):
    Ti = out_ref.shape[0]
    x = x_ref[...]
    w1s = w1s_ref[...]
    w1n = w1n_ref[...]
    b1 = b1_ref[...]
    t1 = (mask_ref[...][:, :, None] * w1s[None, :, :]
          + m1_ref[...][:, :, None] * w1n[None, :, :])
    for i in range(Ti):
        h = jnp.dot(x, t1[i], preferred_element_type=jnp.float32) + b1
        out_ref[i] = jax.nn.relu(h)


def _stage2_body(h1_ref, A_ref, w2s_ref, w2n_ref, b2_ref, wr_ref, bout_ref,
                 out_ref):
    S, Bt, H = h1_ref.shape
    h3 = h1_ref[...]
    h2 = h3.reshape(S, Bt * H)
    agg = jnp.dot(A_ref[...], h2, preferred_element_type=jnp.float32)
    hr = h3.reshape(S * Bt, H)
    ar = agg.reshape(S * Bt, H)
    z = jax.nn.relu(
        jnp.dot(hr, w2s_ref[...], preferred_element_type=jnp.float32)
        + jnp.dot(ar, w2n_ref[...], preferred_element_type=jnp.float32)
        + b2_ref[...])
    z3 = z.reshape(S, Bt, H)
    y = jnp.sum(z3 * wr_ref[...][:, None, :], axis=(0, 2))
    out_ref[...] = y.reshape(Bt, 1) + bout_ref[0, 0]


def kernel(state, action, edge_index, W1_self, W1_neigh, b1,
           W2_self, W2_neigh, b2, W_out, b_out):
    B = state.shape[0]
    S, H = W1_self.shape
    x = jnp.concatenate([state, action], axis=1)
    edgeT = edge_index.T

    A, mask, M1 = pl.pallas_call(
        _prep_body,
        out_shape=[jax.ShapeDtypeStruct((S, S), jnp.float32)] * 3,
    )(edge_index, edgeT)

    Ti = 16
    full = lambda shape: pl.BlockSpec(shape, lambda i: (0,) * len(shape))
    h1 = pl.pallas_call(
        _stage1_body,
        grid=(S // Ti,),
        in_specs=[
            full((B, S)),
            pl.BlockSpec((Ti, S), lambda i: (i, 0)),
            pl.BlockSpec((Ti, S), lambda i: (i, 0)),
            full((S, H)), full((S, H)), full((1, H)),
        ],
        out_specs=pl.BlockSpec((Ti, B, H), lambda i: (i, 0, 0)),
        out_shape=jax.ShapeDtypeStruct((S, B, H), jnp.float32),
    )(x, mask, M1, W1_self, W1_neigh, b1.reshape(1, H))

    Bt = 64
    y = pl.pallas_call(
        _stage2_body,
        grid=(B // Bt,),
        in_specs=[
            pl.BlockSpec((S, Bt, H), lambda j: (0, j, 0)),
            full((S, S)),
            full((H, H)), full((H, H)), full((1, H)),
            full((S, H)), full((1, 1)),
        ],
        out_specs=pl.BlockSpec((Bt, 1), lambda j: (j, 0)),
        out_shape=jax.ShapeDtypeStruct((B, 1), jnp.float32),
    )(h1, A, W2_self, W2_neigh, b2.reshape(1, H), W_out.reshape(S, H),
      b_out.reshape(1, 1))
    return y
